# Initial kernel scaffold; baseline (speedup 1.0000x reference)
import jax
import jax.numpy as jnp
from jax.experimental import pallas as pl


def kernel(x, y):
    raise NotImplementedError("write your pallas kernel here")


# speedup vs baseline: 28.0961x; 28.0961x over previous
import functools

import jax
import jax.numpy as jnp
from jax import lax
from jax.experimental import pallas as pl
from jax.experimental.pallas import tpu as pltpu

_N = 65536
_DIM = 64
_C = 128
_K = 32
_EPS = 1e-6
_MARGIN = 1.0
_B = 256
_NB = _N // _B
_NEG = -1e30
_CHUNK = 256

_INTERPRET = False


def _tc_body(x_ref, y2_ref, y0_ref, out_ref,
             csum_ref, cnt_ref, ut_ref, un_ref,
             cp_ref, cn_ref, dp_ref, dn_ref):
  p = pl.program_id(0)
  b = pl.program_id(1)
  lanes = lax.broadcasted_iota(jnp.int32, (1, _C), 1)

  @pl.when(jnp.logical_and(p == 0, b == 0))
  def _init():
    csum_ref[...] = jnp.zeros((_DIM, _C), jnp.float32)
    cnt_ref[...] = jnp.zeros((1, _C), jnp.float32)

  @pl.when(p == 0)
  def _accum():
    x_blk = x_ref[...]
    oh = (y2_ref[...] == lanes).astype(jnp.float32)
    csum_ref[...] += lax.dot_general(
        x_blk, oh, (((0,), (0,)), ((), ())),
        preferred_element_type=jnp.float32)
    cnt_ref[...] += jnp.sum(oh, axis=0, keepdims=True)

  @pl.when(jnp.logical_and(p == 1, b == 0))
  def _centers():
    ut = csum_ref[...] / cnt_ref[...] + _EPS
    ut_ref[...] = ut
    un_ref[...] = jnp.sum(ut * ut, axis=0, keepdims=True)

  @pl.when(p == 1)
  def _distance_block():
    x_blk = x_ref[...]
    cross = lax.dot_general(
        x_blk, ut_ref[...], (((1,), (0,)), ((), ())),
        preferred_element_type=jnp.float32)
    xnorm = jnp.sum(x_blk * x_blk, axis=1, keepdims=True)
    d = jnp.sqrt(jnp.maximum(un_ref[...] - 2.0 * cross + xnorm, 0.0))

    ppos = jnp.where(y2_ref[...] == lanes, d, _NEG)
    pneg = jnp.where(y0_ref[...] != lanes, -d, _NEG)
    base = b * _K

    def extract(k, pv):
      m = jnp.max(pv, axis=0, keepdims=True)
      return m, jnp.where(pv == m, _NEG, pv)

    def pos_body(k, pv):
      m, pv = extract(k, pv)
      cp_ref[pl.ds(base + k, 1), :] = m
      return pv

    def neg_body(k, pv):
      m, pv = extract(k, pv)
      cn_ref[pl.ds(base + k, 1), :] = m
      return pv

    lax.fori_loop(0, _K, pos_body, ppos)
    lax.fori_loop(0, _K, neg_body, pneg)

  @pl.when(jnp.logical_and(p == 1, b == _NB - 1))
  def _finish():
    nrows = _NB * _K
    nchunks = nrows // _CHUNK

    def merge(src_ref, dst_ref):
      def kbody(k, _):
        def cmax(c, m):
          blk = src_ref[pl.ds(c * _CHUNK, _CHUNK), :]
          return jnp.maximum(m, jnp.max(blk, axis=0, keepdims=True))
        m = lax.fori_loop(0, nchunks, cmax, jnp.full((1, _C), _NEG, jnp.float32))
        dst_ref[pl.ds(k, 1), :] = m

        def cupd(c, _):
          blk = src_ref[pl.ds(c * _CHUNK, _CHUNK), :]
          src_ref[pl.ds(c * _CHUNK, _CHUNK), :] = jnp.where(blk == m, _NEG, blk)
          return 0
        lax.fori_loop(0, nchunks, cupd, 0)
        return 0
      lax.fori_loop(0, _K, kbody, 0)

    merge(cp_ref, dp_ref)
    merge(cn_ref, dn_ref)

    hinge = jnp.maximum(dp_ref[...] + dn_ref[...] + _MARGIN, 0.0)
    lo = jnp.sum(hinge, axis=0, keepdims=True) / float(_K)
    present = cnt_ref[...] > 0.0
    lo = jnp.where(present, lo, 0.0)
    n_present = jnp.sum(present.astype(jnp.float32))
    out_ref[...] = (jnp.sum(lo, axis=1, keepdims=True) / n_present)


@functools.partial(jax.jit, static_argnames=())
def kernel(x, y):
  y2 = y[:, 2:3]
  y0 = y[:, 0:1]
  out = pl.pallas_call(
      _tc_body,
      grid=(2, _NB),
      in_specs=[
          pl.BlockSpec((_B, _DIM), lambda p, b: (b, 0)),
          pl.BlockSpec((_B, 1), lambda p, b: (b, 0)),
          pl.BlockSpec((_B, 1), lambda p, b: (b, 0)),
      ],
      out_specs=pl.BlockSpec((1, 1), lambda p, b: (0, 0)),
      out_shape=jax.ShapeDtypeStruct((1, 1), jnp.float32),
      scratch_shapes=[
          pltpu.VMEM((_DIM, _C), jnp.float32),
          pltpu.VMEM((1, _C), jnp.float32),
          pltpu.VMEM((_DIM, _C), jnp.float32),
          pltpu.VMEM((1, _C), jnp.float32),
          pltpu.VMEM((_NB * _K, _C), jnp.float32),
          pltpu.VMEM((_NB * _K, _C), jnp.float32),
          pltpu.VMEM((_K, _C), jnp.float32),
          pltpu.VMEM((_K, _C), jnp.float32),
      ],
      compiler_params=pltpu.CompilerParams(
          dimension_semantics=("arbitrary", "arbitrary")),
      interpret=_INTERPRET,
  )(x, y2, y0)
  return out.reshape((1,))


# speedup vs baseline: 36.4118x; 1.2960x over previous
import functools

import jax
import jax.numpy as jnp
from jax import lax
from jax.experimental import pallas as pl
from jax.experimental.pallas import tpu as pltpu

_N = 65536
_DIM = 64
_C = 128
_K = 32
_KB = 8
_EPS = 1e-6
_MARGIN = 1.0
_B = 256
_NB = _N // _B
_NEG = -1e30
_CHUNK = 256

_INTERPRET = False


def _tc_body(x_ref, y2_ref, y0_ref, out_ref,
             csum_ref, cnt_ref, ut_ref, un_ref,
             cp_ref, cn_ref, mp_ref, mn_ref,
             bp_ref, bn_ref, dp_ref, dn_ref, trig_ref):
  p = pl.program_id(0)
  b = pl.program_id(1)
  lanes = lax.broadcasted_iota(jnp.int32, (1, _C), 1)

  def block_d2():
    x_blk = x_ref[...]
    cross = lax.dot_general(
        x_blk, ut_ref[...], (((1,), (0,)), ((), ())),
        preferred_element_type=jnp.float32)
    xnorm = jnp.sum(x_blk * x_blk, axis=1, keepdims=True)
    d2 = jnp.maximum(un_ref[...] - 2.0 * cross + xnorm, 0.0)
    ppos = jnp.where(y2_ref[...] == lanes, d2, _NEG)
    pneg = jnp.where(y0_ref[...] != lanes, -d2, _NEG)
    return ppos, pneg

  def extract(dst_ref, base, depth, pv):
    def body(k, carry):
      pv, _ = carry
      m = jnp.max(pv, axis=0, keepdims=True)
      dst_ref[pl.ds(base + k, 1), :] = m
      return jnp.where(pv == m, _NEG, pv), m
    _, last = lax.fori_loop(0, depth, body,
                            (pv, jnp.full((1, _C), _NEG, jnp.float32)))
    return last

  def merge(src_ref, nrows, dst_ref):
    nchunks = nrows // _CHUNK
    def kbody(k, _):
      def cmax(c, m):
        blk = src_ref[pl.ds(c * _CHUNK, _CHUNK), :]
        return jnp.maximum(m, jnp.max(blk, axis=0, keepdims=True))
      m = lax.fori_loop(0, nchunks, cmax, jnp.full((1, _C), _NEG, jnp.float32))
      dst_ref[pl.ds(k, 1), :] = m
      def cupd(c, _):
        blk = src_ref[pl.ds(c * _CHUNK, _CHUNK), :]
        src_ref[pl.ds(c * _CHUNK, _CHUNK), :] = jnp.where(blk == m, _NEG, blk)
        return 0
      lax.fori_loop(0, nchunks, cupd, 0)
      return 0
    lax.fori_loop(0, _K, kbody, 0)

  def emit_loss():
    d_ap = jnp.sqrt(jnp.maximum(dp_ref[...], 0.0))
    d_an = jnp.sqrt(jnp.maximum(-dn_ref[...], 0.0))
    hinge = jnp.maximum(d_ap - d_an + _MARGIN, 0.0)
    lo = jnp.sum(hinge, axis=0, keepdims=True) / float(_K)
    present = cnt_ref[...] > 0.0
    lo = jnp.where(present, lo, 0.0)
    n_present = jnp.sum(present.astype(jnp.float32))
    out_ref[...] = (jnp.sum(lo, axis=1, keepdims=True) / n_present)

  @pl.when(jnp.logical_and(p == 0, b == 0))
  def _init():
    csum_ref[...] = jnp.zeros((_DIM, _C), jnp.float32)
    cnt_ref[...] = jnp.zeros((1, _C), jnp.float32)

  @pl.when(p == 0)
  def _accum():
    oh = (y2_ref[...] == lanes).astype(jnp.float32)
    csum_ref[...] += lax.dot_general(
        x_ref[...], oh, (((0,), (0,)), ((), ())),
        preferred_element_type=jnp.float32)
    cnt_ref[...] += jnp.sum(oh, axis=0, keepdims=True)

  @pl.when(jnp.logical_and(p == 1, b == 0))
  def _centers():
    ut = csum_ref[...] / cnt_ref[...] + _EPS
    ut_ref[...] = ut
    un_ref[...] = jnp.sum(ut * ut, axis=0, keepdims=True)

  @pl.when(p == 1)
  def _distance_block():
    ppos, pneg = block_d2()
    mp_ref[pl.ds(b, 1), :] = extract(cp_ref, b * _KB, _KB, ppos)
    mn_ref[pl.ds(b, 1), :] = extract(cn_ref, b * _KB, _KB, pneg)

  @pl.when(jnp.logical_and(p == 1, b == _NB - 1))
  def _fast_finish():
    merge(cp_ref, _NB * _KB, dp_ref)
    merge(cn_ref, _NB * _KB, dn_ref)
    live = lanes < 100
    t32p = dp_ref[pl.ds(_K - 1, 1), :]
    t32n = dn_ref[pl.ds(_K - 1, 1), :]
    trig_p = jnp.any((mp_ref[...] >= t32p) & (t32p > -1e29) & live)
    trig_n = jnp.any((mn_ref[...] >= t32n) & (t32n > -1e29) & live)
    trig_ref[0] = jnp.logical_or(trig_p, trig_n).astype(jnp.int32)
    emit_loss()

  @pl.when(jnp.logical_and(p == 2, trig_ref[0] != 0))
  def _slow_block():
    ppos, pneg = block_d2()
    extract(bp_ref, b * _K, _K, ppos)
    extract(bn_ref, b * _K, _K, pneg)

  @pl.when(jnp.logical_and(jnp.logical_and(p == 2, b == _NB - 1),
                           trig_ref[0] != 0))
  def _slow_finish():
    merge(bp_ref, _NB * _K, dp_ref)
    merge(bn_ref, _NB * _K, dn_ref)
    emit_loss()


@functools.partial(jax.jit, static_argnames=())
def kernel(x, y):
  y2 = y[:, 2:3]
  y0 = y[:, 0:1]
  out = pl.pallas_call(
      _tc_body,
      grid=(3, _NB),
      in_specs=[
          pl.BlockSpec((_B, _DIM), lambda p, b: (b, 0)),
          pl.BlockSpec((_B, 1), lambda p, b: (b, 0)),
          pl.BlockSpec((_B, 1), lambda p, b: (b, 0)),
      ],
      out_specs=pl.BlockSpec((1, 1), lambda p, b: (0, 0)),
      out_shape=jax.ShapeDtypeStruct((1, 1), jnp.float32),
      scratch_shapes=[
          pltpu.VMEM((_DIM, _C), jnp.float32),
          pltpu.VMEM((1, _C), jnp.float32),
          pltpu.VMEM((_DIM, _C), jnp.float32),
          pltpu.VMEM((1, _C), jnp.float32),
          pltpu.VMEM((_NB * _KB, _C), jnp.float32),
          pltpu.VMEM((_NB * _KB, _C), jnp.float32),
          pltpu.VMEM((_NB, _C), jnp.float32),
          pltpu.VMEM((_NB, _C), jnp.float32),
          pltpu.VMEM((_NB * _K, _C), jnp.float32),
          pltpu.VMEM((_NB * _K, _C), jnp.float32),
          pltpu.VMEM((_K, _C), jnp.float32),
          pltpu.VMEM((_K, _C), jnp.float32),
          pltpu.SMEM((1,), jnp.int32),
      ],
      compiler_params=pltpu.CompilerParams(
          dimension_semantics=("arbitrary", "arbitrary")),
      interpret=_INTERPRET,
  )(x, y2, y0)
  return out.reshape((1,))
